# Initial kernel scaffold; baseline (speedup 1.0000x reference)
import jax
import jax.numpy as jnp
from jax.experimental import pallas as pl


def kernel(x, seeds, Wq):
    raise NotImplementedError("write your pallas kernel here")


# speedup vs baseline: 6.0730x; 6.0730x over previous
import functools
import math

import jax
import jax.numpy as jnp
from jax.experimental import pallas as pl
from jax.experimental.pallas import tpu as pltpu

_D = 128
_NSEEDS = 500
_NPAD = 512
_K = 4
_BBLK = 512
_NEG = -1e30


def _body(x_ref, seeds_ref, wq_ref, field_ref, attn_ref):
    x = x_ref[...]
    seeds = seeds_ref[...]
    wq = wq_ref[...]
    q = jax.lax.dot_general(x, wq, (((1,), (1,)), ((), ())),
                            preferred_element_type=jnp.float32)
    s = jax.lax.dot_general(q, seeds, (((1,), (1,)), ((), ())),
                            preferred_element_type=jnp.float32)
    s = s * (1.0 / math.sqrt(_D))
    cols = jax.lax.broadcasted_iota(jnp.int32, (_BBLK, _NPAD), 1)
    s = jnp.where(cols < _NSEEDS, s, _NEG)

    top_v = []
    top_i = []
    for _ in range(_K):
        amax = jnp.max(s, axis=1, keepdims=True)
        hit = s == amax
        idx = jnp.min(jnp.where(hit, cols, _NPAD), axis=1, keepdims=True)
        top_v.append(amax)
        top_i.append(idx)
        s = jnp.where(cols == idx, _NEG, s)

    tv = jnp.concatenate(top_v, axis=1)
    attn = jnp.exp(tv - tv[:, 0:1])
    attn = attn / jnp.sum(attn, axis=1, keepdims=True)
    attn_ref[...] = attn

    w = jnp.zeros((_BBLK, _NPAD), jnp.float32)
    for k in range(_K):
        w = w + jnp.where(cols == top_i[k], attn[:, k:k + 1], 0.0)
    field_ref[...] = jax.lax.dot_general(
        w, seeds, (((1,), (0,)), ((), ())),
        preferred_element_type=jnp.float32)


def kernel(x, seeds, Wq):
    batch = x.shape[0]
    seeds_p = jnp.zeros((_NPAD, _D), jnp.float32).at[:_NSEEDS].set(seeds)
    grid = (batch // _BBLK,)
    field, attn = pl.pallas_call(
        _body,
        grid=grid,
        in_specs=[
            pl.BlockSpec((_BBLK, _D), lambda i: (i, 0)),
            pl.BlockSpec((_NPAD, _D), lambda i: (0, 0)),
            pl.BlockSpec((_D, _D), lambda i: (0, 0)),
        ],
        out_specs=[
            pl.BlockSpec((_BBLK, _D), lambda i: (i, 0)),
            pl.BlockSpec((_BBLK, _K), lambda i: (i, 0)),
        ],
        out_shape=[
            jax.ShapeDtypeStruct((batch, _D), jnp.float32),
            jax.ShapeDtypeStruct((batch, _K), jnp.float32),
        ],
    )(x, seeds_p, Wq)
    return (field, attn)


# speedup vs baseline: 14.0659x; 2.3161x over previous
import math

import jax
import jax.numpy as jnp
from jax.experimental import pallas as pl

_D = 128
_NSEEDS = 500
_NPAD = 512
_K = 4
_BBLK = 512
_NEG = -1e30


def _body(x_ref, seeds_ref, wq_ref, field_ref, attn_ref):
    x = x_ref[...]
    seeds = seeds_ref[...]
    wq = wq_ref[...]
    q = jax.lax.dot_general(x, wq, (((1,), (1,)), ((), ())),
                            preferred_element_type=jnp.float32)
    s = jax.lax.dot_general(q, seeds, (((1,), (1,)), ((), ())),
                            preferred_element_type=jnp.float32)
    s = s * (1.0 / math.sqrt(_D))
    cols = jax.lax.broadcasted_iota(jnp.int32, (_BBLK, _NPAD), 1)
    s = jnp.where(cols < _NSEEDS, s, _NEG)

    v1 = jnp.max(s, axis=1, keepdims=True)
    hit = s == v1
    w = jnp.where(hit, 1.0, 0.0)
    s = jnp.where(hit, _NEG, s)
    tv = [v1]
    ev = [jnp.ones_like(v1)]
    for k in range(1, _K):
        vk = jnp.max(s, axis=1, keepdims=True)
        hit = s == vk
        e = jnp.exp(vk - v1)
        w = w + jnp.where(hit, e, 0.0)
        if k < _K - 1:
            s = jnp.where(hit, _NEG, s)
        tv.append(vk)
        ev.append(e)

    z = ev[0] + ev[1] + ev[2] + ev[3]
    rz = 1.0 / z
    attn_ref[...] = jnp.concatenate(ev, axis=1) * rz

    f = jax.lax.dot_general(w, seeds, (((1,), (0,)), ((), ())),
                            preferred_element_type=jnp.float32)
    field_ref[...] = f * rz


def kernel(x, seeds, Wq):
    batch = x.shape[0]
    seeds_p = jnp.zeros((_NPAD, _D), jnp.float32).at[:_NSEEDS].set(seeds)
    grid = (batch // _BBLK,)
    field, attn = pl.pallas_call(
        _body,
        grid=grid,
        in_specs=[
            pl.BlockSpec((_BBLK, _D), lambda i: (i, 0)),
            pl.BlockSpec((_NPAD, _D), lambda i: (0, 0)),
            pl.BlockSpec((_D, _D), lambda i: (0, 0)),
        ],
        out_specs=[
            pl.BlockSpec((_BBLK, _D), lambda i: (i, 0)),
            pl.BlockSpec((_BBLK, _K), lambda i: (i, 0)),
        ],
        out_shape=[
            jax.ShapeDtypeStruct((batch, _D), jnp.float32),
            jax.ShapeDtypeStruct((batch, _K), jnp.float32),
        ],
    )(x, seeds_p, Wq)
    return (field, attn)


# speedup vs baseline: 14.7438x; 1.0482x over previous
import math

import jax
import jax.numpy as jnp
from jax.experimental import pallas as pl

_D = 128
_NSEEDS = 500
_NPAD = 512
_K = 4
_BBLK = 512
_NEG = -1e30


def _body(x_ref, seeds_ref, wq_ref, mrow_ref, field_ref, attn_ref):
    x = x_ref[...]
    seeds = seeds_ref[...]
    wq = wq_ref[...]
    q = jax.lax.dot_general(x, wq, (((1,), (1,)), ((), ())),
                            preferred_element_type=jnp.float32)
    s = jax.lax.dot_general(q, seeds, (((1,), (1,)), ((), ())),
                            preferred_element_type=jnp.float32)
    s = s + mrow_ref[0:1, :]

    v1 = jnp.max(s, axis=1, keepdims=True)
    s1 = jnp.where(s >= v1, _NEG, s)
    v2 = jnp.max(s1, axis=1, keepdims=True)
    s2 = jnp.where(s1 >= v2, _NEG, s1)
    v3 = jnp.max(s2, axis=1, keepdims=True)
    s3 = jnp.where(s2 >= v3, _NEG, s2)
    v4 = jnp.max(s3, axis=1, keepdims=True)

    e2 = jnp.exp(v2 - v1)
    e3 = jnp.exp(v3 - v1)
    e4 = jnp.exp(v4 - v1)
    rz = 1.0 / (1.0 + e2 + e3 + e4)
    attn_ref[...] = jnp.concatenate([jnp.ones_like(v1), e2, e3, e4],
                                    axis=1) * rz

    w = jnp.exp(jnp.where(s >= v4, s - v1, _NEG))
    f = jax.lax.dot_general(w, seeds, (((1,), (0,)), ((), ())),
                            preferred_element_type=jnp.float32)
    field_ref[...] = f * rz


def kernel(x, seeds, Wq):
    batch = x.shape[0]
    seeds_p = jnp.zeros((_NPAD, _D), jnp.float32).at[:_NSEEDS].set(seeds)
    wq_s = Wq * (1.0 / math.sqrt(_D))
    mrow = jnp.where(jnp.arange(_NPAD)[None, :] < _NSEEDS, 0.0, _NEG)
    mrow = jnp.broadcast_to(mrow, (8, _NPAD)).astype(jnp.float32)
    grid = (batch // _BBLK,)
    field, attn = pl.pallas_call(
        _body,
        grid=grid,
        in_specs=[
            pl.BlockSpec((_BBLK, _D), lambda i: (i, 0)),
            pl.BlockSpec((_NPAD, _D), lambda i: (0, 0)),
            pl.BlockSpec((_D, _D), lambda i: (0, 0)),
            pl.BlockSpec((8, _NPAD), lambda i: (0, 0)),
        ],
        out_specs=[
            pl.BlockSpec((_BBLK, _D), lambda i: (i, 0)),
            pl.BlockSpec((_BBLK, _K), lambda i: (i, 0)),
        ],
        out_shape=[
            jax.ShapeDtypeStruct((batch, _D), jnp.float32),
            jax.ShapeDtypeStruct((batch, _K), jnp.float32),
        ],
    )(x, seeds_p, wq_s, mrow)
    return (field, attn)


# speedup vs baseline: 15.2674x; 1.0355x over previous
import math

import jax
import jax.numpy as jnp
from jax.experimental import pallas as pl

_D = 128
_NSEEDS = 500
_NPAD = 512
_K = 4
_BBLK = 512
_NEG = -1e30


def _body(x_ref, seeds_ref, wq_ref, field_ref, attn_ref):
    x = x_ref[...]
    seeds = seeds_ref[...]
    wq = wq_ref[...]
    q = jax.lax.dot_general(x, wq, (((1,), (1,)), ((), ())),
                            preferred_element_type=jnp.float32)
    s = jax.lax.dot_general(q, seeds, (((1,), (1,)), ((), ())),
                            preferred_element_type=jnp.float32)
    s = s * (1.0 / math.sqrt(_D))
    cols = jax.lax.broadcasted_iota(jnp.int32, (_BBLK, _NPAD), 1)
    s = jnp.where(cols < _NSEEDS, s, _NEG)

    v1 = jnp.max(s, axis=1, keepdims=True)
    s1 = jnp.where(s >= v1, _NEG, s)
    v2 = jnp.max(s1, axis=1, keepdims=True)
    s2 = jnp.where(s1 >= v2, _NEG, s1)
    v3 = jnp.max(s2, axis=1, keepdims=True)
    s3 = jnp.where(s2 >= v3, _NEG, s2)
    v4 = jnp.max(s3, axis=1, keepdims=True)

    e2 = jnp.exp(v2 - v1)
    e3 = jnp.exp(v3 - v1)
    e4 = jnp.exp(v4 - v1)
    rz = 1.0 / (1.0 + e2 + e3 + e4)
    attn_ref[...] = jnp.concatenate([jnp.ones_like(v1), e2, e3, e4],
                                    axis=1) * rz

    w = jnp.exp(jnp.where(s >= v4, s - v1, -100.0))
    f = jax.lax.dot_general(w, seeds, (((1,), (0,)), ((), ())),
                            preferred_element_type=jnp.float32)
    field_ref[...] = f * rz


def kernel(x, seeds, Wq):
    batch = x.shape[0]
    seeds_p = jnp.zeros((_NPAD, _D), jnp.float32).at[:_NSEEDS].set(seeds)
    grid = (batch // _BBLK,)
    field, attn = pl.pallas_call(
        _body,
        grid=grid,
        in_specs=[
            pl.BlockSpec((_BBLK, _D), lambda i: (i, 0)),
            pl.BlockSpec((_NPAD, _D), lambda i: (0, 0)),
            pl.BlockSpec((_D, _D), lambda i: (0, 0)),
        ],
        out_specs=[
            pl.BlockSpec((_BBLK, _D), lambda i: (i, 0)),
            pl.BlockSpec((_BBLK, _K), lambda i: (i, 0)),
        ],
        out_shape=[
            jax.ShapeDtypeStruct((batch, _D), jnp.float32),
            jax.ShapeDtypeStruct((batch, _K), jnp.float32),
        ],
    )(x, seeds_p, Wq)
    return (field, attn)
